# Initial kernel scaffold; baseline (speedup 1.0000x reference)
import jax
import jax.numpy as jnp
from jax.experimental import pallas as pl


def kernel(inputs, table):
    raise NotImplementedError("write your pallas kernel here")


# speedup vs baseline: 2.3715x; 2.3715x over previous
import functools

import jax
import jax.numpy as jnp
from jax import lax
from jax.experimental import pallas as pl
from jax.experimental.pallas import tpu as pltpu
from jax.experimental.pallas import tpu_sc as plsc

_D = 1024
_B = 4 * 8192
_NC = 2
_NS = 16
_NW = _NC * _NS
_BPW = _B // _NW
_CH = 32
_NCH = _BPW // _CH


def _make_gather():
    mesh = plsc.VectorSubcoreMesh(core_axis_name="c", subcore_axis_name="s")

    @functools.partial(
        pl.kernel,
        mesh=mesh,
        out_type=jax.ShapeDtypeStruct((_B, _D), jnp.float32),
        scratch_types=[
            pltpu.VMEM((_NCH, _CH), jnp.int32),
            pltpu.VMEM((_CH, _D), jnp.float32),
            pltpu.VMEM((_CH, _D), jnp.float32),
            pltpu.SemaphoreType.DMA,
            pltpu.SemaphoreType.DMA,
        ],
    )
    def gather(idx_hbm, table_hbm, out_hbm, idx_v, buf0, buf1, sem0, sem1):
        wid = lax.axis_index("s") * _NC + lax.axis_index("c")
        base = wid * _BPW
        pltpu.sync_copy(idx_hbm.at[wid], idx_v)
        pltpu.async_copy(table_hbm.at[idx_v.at[0]], buf0, sem0)

        def body(i, carry):
            j = 2 * i
            pltpu.async_copy(table_hbm.at[idx_v.at[j + 1]], buf1, sem1)
            pltpu.make_async_copy(
                table_hbm.at[idx_v.at[j]], buf0, sem0).wait()
            pltpu.sync_copy(buf0, out_hbm.at[pl.ds(base + j * _CH, _CH)])

            @pl.when(j + 2 < _NCH)
            def _():
                pltpu.async_copy(table_hbm.at[idx_v.at[j + 2]], buf0, sem0)

            pltpu.make_async_copy(
                table_hbm.at[idx_v.at[j + 1]], buf1, sem1).wait()
            pltpu.sync_copy(buf1, out_hbm.at[pl.ds(base + (j + 1) * _CH, _CH)])
            return carry

        lax.fori_loop(0, _NCH // 2, body, 0)

    return gather


_gather = _make_gather()


@jax.jit
def kernel(inputs, table):
    idx = inputs.reshape(_NW, _NCH, _CH)
    out = _gather(idx, table)
    return out.reshape(inputs.shape + (_D,))


# speedup vs baseline: 2.3872x; 1.0066x over previous
import functools

import jax
import jax.numpy as jnp
from jax import lax
from jax.experimental import pallas as pl
from jax.experimental.pallas import tpu as pltpu
from jax.experimental.pallas import tpu_sc as plsc

_D = 1024
_B = 4 * 8192
_NC = 2
_NS = 16
_NW = _NC * _NS
_BPW = _B // _NW
_CH = 16
_NCH = _BPW // _CH
_NBUF = 4


def _make_gather():
    mesh = plsc.VectorSubcoreMesh(core_axis_name="c", subcore_axis_name="s")

    @functools.partial(
        pl.kernel,
        mesh=mesh,
        out_type=jax.ShapeDtypeStruct((_B, _D), jnp.float32),
        scratch_types=[
            pltpu.VMEM((_NCH, _CH), jnp.int32),
            *([pltpu.VMEM((_CH, _D), jnp.float32)] * _NBUF),
            *([pltpu.SemaphoreType.DMA] * (2 * _NBUF)),
        ],
    )
    def gather(idx_hbm, table_hbm, out_hbm, idx_v, *bufs_and_sems):
        bufs = bufs_and_sems[:_NBUF]
        gsem = bufs_and_sems[_NBUF:2 * _NBUF]
        ssem = bufs_and_sems[2 * _NBUF:]
        wid = lax.axis_index("s") * _NC + lax.axis_index("c")
        base = wid * _BPW

        def gather_chunk(c, b):
            return pltpu.make_async_copy(
                table_hbm.at[idx_v.at[c]], bufs[b], gsem[b])

        def store_chunk(c, b):
            return pltpu.make_async_copy(
                bufs[b], out_hbm.at[pl.ds(base + c * _CH, _CH)], ssem[b])

        pltpu.sync_copy(idx_hbm.at[wid], idx_v)
        gather_chunk(0, 0).start()
        gather_chunk(1, 1).start()

        def body(g, carry):
            for b in range(_NBUF):
                c = _NBUF * g + b
                gather_chunk(c, b).wait()
                store_chunk(c, b).start()
                n = c + 2
                bn = (b + 2) % _NBUF

                @pl.when((n >= _NBUF) & (n < _NCH))
                def _():
                    store_chunk(n - _NBUF, bn).wait()

                @pl.when(n < _NCH)
                def _():
                    gather_chunk(n, bn).start()
            return carry

        lax.fori_loop(0, _NCH // _NBUF, body, 0)
        for k in range(_NBUF):
            c = _NCH - _NBUF + k
            store_chunk(c, c % _NBUF).wait()

    return gather


_gather = _make_gather()


@jax.jit
def kernel(inputs, table):
    idx = inputs.reshape(_NW, _NCH, _CH)
    out = _gather(idx, table)
    return out.reshape(inputs.shape + (_D,))
